# baseline (speedup 1.0000x reference)
import functools

import jax
import jax.numpy as jnp
from jax import lax
from jax.experimental import pallas as pl
from jax.experimental.pallas import tpu as pltpu
from jax.experimental.pallas import tpu_sc as plsc

_NC = 2
_NS = 16
_NW = _NC * _NS

VOCAB = 100000
DIM = 300
BATCH = 1024
SEQ = 50
HEAD = 256
TAIL = DIM - HEAD
SHIFT = DIM - 128
B_PER_W = BATCH // _NW
_NBUF = 2

mesh = plsc.VectorSubcoreMesh(core_axis_name="c", subcore_axis_name="s")


@functools.partial(
    pl.kernel,
    mesh=mesh,
    out_type=(
        jax.ShapeDtypeStruct((BATCH, SEQ, DIM), jnp.float32),
        jax.ShapeDtypeStruct((BATCH, SEQ, 128), jnp.float32),
    ),
    scratch_types=[
        pltpu.VMEM((B_PER_W, SEQ), jnp.int32),
        pltpu.VMEM((_NBUF, SEQ, 128), jnp.float32),
        pltpu.VMEM((_NBUF, SEQ, 128), jnp.float32),
        pltpu.VMEM((_NBUF, SEQ, 128), jnp.float32),
        pltpu.SemaphoreType.DMA((_NBUF,)),
    ],
)
def _gather_sc(idx_hbm, table_hbm, out_hbm, tail_hbm,
               idx_v, buf_a, buf_b, buf_t, sems):
    wid = lax.axis_index("s") * _NC + lax.axis_index("c")
    b0 = wid * B_PER_W
    pltpu.sync_copy(idx_hbm.at[pl.ds(b0, B_PER_W)], idx_v)

    def start(c, buf):
        pltpu.async_copy(
            table_hbm.at[idx_v.at[c], pl.ds(0, 128)], buf_a.at[buf],
            sems.at[buf],
        )
        pltpu.async_copy(
            table_hbm.at[idx_v.at[c], pl.ds(128, 128)], buf_b.at[buf],
            sems.at[buf],
        )
        pltpu.async_copy(
            table_hbm.at[idx_v.at[c], pl.ds(256, 128)], buf_t.at[buf],
            sems.at[buf],
        )

    def wait(c, buf):
        pltpu.make_async_copy(
            table_hbm.at[idx_v.at[c], pl.ds(0, 128)], buf_a.at[buf],
            sems.at[buf],
        ).wait()
        pltpu.make_async_copy(
            table_hbm.at[idx_v.at[c], pl.ds(128, 128)], buf_b.at[buf],
            sems.at[buf],
        ).wait()
        pltpu.make_async_copy(
            table_hbm.at[idx_v.at[c], pl.ds(256, 128)], buf_t.at[buf],
            sems.at[buf],
        ).wait()

    for b in range(_NBUF):
        start(b, b)

    @pl.loop(0, B_PER_W, step=_NBUF)
    def _chunks(g):
        for b in range(_NBUF):
            c = g + b
            wait(c, b)
            pltpu.sync_copy(buf_a.at[b], out_hbm.at[b0 + c, :, pl.ds(0, 128)])
            pltpu.sync_copy(buf_b.at[b], out_hbm.at[b0 + c, :, pl.ds(128, 128)])
            pltpu.sync_copy(buf_t.at[b], tail_hbm.at[b0 + c])

            @pl.when(c + _NBUF < B_PER_W)
            def _():
                start(c + _NBUF, b)


@jax.jit
def kernel(vocab_id_list, table):
    tablep = jnp.pad(table, ((0, 0), (0, 384 - DIM)))
    out, tail = _gather_sc(vocab_id_list, tablep)
    tail44 = lax.slice(tail, (0, 0, 0), (BATCH, SEQ, TAIL))
    return lax.dynamic_update_slice(out, tail44, (0, 0, HEAD))


# speedup vs baseline: 2.2684x; 2.2684x over previous
import functools

import jax
import jax.numpy as jnp
from jax import lax
from jax.experimental import pallas as pl
from jax.experimental.pallas import tpu as pltpu
from jax.experimental.pallas import tpu_sc as plsc

_NC = 2
_NS = 16
_NW = _NC * _NS

VOCAB = 100000
DIM = 300
BATCH = 1024
SEQ = 50
HEAD = 256
TAIL = DIM - HEAD
SHIFT = DIM - 128
B_PER_W = BATCH // _NW
_NBUF = 4

mesh = plsc.VectorSubcoreMesh(core_axis_name="c", subcore_axis_name="s")


@functools.partial(
    pl.kernel,
    mesh=mesh,
    out_type=(
        jax.ShapeDtypeStruct((BATCH, SEQ, DIM), jnp.float32),
        jax.ShapeDtypeStruct((BATCH, SEQ, 128), jnp.float32),
    ),
    scratch_types=[
        pltpu.VMEM((B_PER_W, SEQ), jnp.int32),
        pltpu.VMEM((_NBUF, SEQ, 128), jnp.float32),
        pltpu.VMEM((_NBUF, SEQ, 128), jnp.float32),
        pltpu.VMEM((_NBUF, SEQ, 128), jnp.float32),
        pltpu.SemaphoreType.DMA((_NBUF,)),
    ],
)
def _gather_sc(idx_hbm, table_hbm, tailsrc_hbm, out_hbm, tail_hbm,
               idx_v, buf_a, buf_b, buf_t, sems):
    wid = lax.axis_index("s") * _NC + lax.axis_index("c")
    b0 = wid * B_PER_W
    pltpu.sync_copy(idx_hbm.at[pl.ds(b0, B_PER_W)], idx_v)

    def start(c, buf):
        pltpu.async_copy(
            table_hbm.at[idx_v.at[c], pl.ds(0, 128)], buf_a.at[buf],
            sems.at[buf],
        )
        pltpu.async_copy(
            table_hbm.at[idx_v.at[c], pl.ds(128, 128)], buf_b.at[buf],
            sems.at[buf],
        )
        pltpu.async_copy(
            tailsrc_hbm.at[idx_v.at[c]], buf_t.at[buf], sems.at[buf],
        )

    def wait(c, buf):
        pltpu.make_async_copy(
            table_hbm.at[idx_v.at[c], pl.ds(0, 128)], buf_a.at[buf],
            sems.at[buf],
        ).wait()
        pltpu.make_async_copy(
            table_hbm.at[idx_v.at[c], pl.ds(128, 128)], buf_b.at[buf],
            sems.at[buf],
        ).wait()
        pltpu.make_async_copy(
            tailsrc_hbm.at[idx_v.at[c]], buf_t.at[buf], sems.at[buf],
        ).wait()

    for b in range(_NBUF):
        start(b, b)

    @pl.loop(0, B_PER_W, step=_NBUF)
    def _chunks(g):
        for b in range(_NBUF):
            c = g + b
            wait(c, b)
            pltpu.sync_copy(buf_a.at[b], out_hbm.at[b0 + c, :, pl.ds(0, 128)])
            pltpu.sync_copy(buf_b.at[b], out_hbm.at[b0 + c, :, pl.ds(128, 128)])
            pltpu.sync_copy(buf_t.at[b], tail_hbm.at[b0 + c])

            @pl.when(c + _NBUF < B_PER_W)
            def _():
                start(c + _NBUF, b)


@jax.jit
def kernel(vocab_id_list, table):
    tailsrc = lax.slice(table, (0, SHIFT), (VOCAB, DIM))
    out, tail = _gather_sc(vocab_id_list, table, tailsrc)
    tail44 = lax.slice(tail, (0, 0, 128 - TAIL), (BATCH, SEQ, 128))
    return lax.dynamic_update_slice(out, tail44, (0, 0, HEAD))
